# baseline (speedup 1.0000x reference)
import functools

import jax
import jax.numpy as jnp
from jax import lax
from jax.experimental import pallas as pl
from jax.experimental.pallas import tpu as pltpu
from jax.experimental.pallas import tpu_sc as plsc

N = 10000
E = 320000
D = 128
H = 64
G = 64

NC = 2
NS = 16
NW = NC * NS
CH = 128
NCHUNK = 80
EPW = NCHUNK * CH
EPAD = NW * EPW
NP = N + 8
IDXB = 10
NBLK = NCHUNK // IDXB
SUB_ROWS = 624
TAIL_BASE = NS * SUB_ROWS
ACC_TAIL = NP - TAIL_BASE
DUMP_TAIL = N - TAIL_BASE

def _sc_aggregate_body(m_hbm, src_hbm, dst_hbm, z_hbm, p_hbm,
                       src_v, dst_v, rows_a, rows_b, acc, sem_a, sem_b):
    c = lax.axis_index("c")
    s = lax.axis_index("s")
    w = s * NC + c
    base = s * SUB_ROWS

    @pl.when(c == 0)
    def _():
        pltpu.sync_copy(m_hbm.at[pl.ds(base, SUB_ROWS)],
                        acc.at[pl.ds(base, SUB_ROWS)])

        @pl.when(s == NS - 1)
        def _():
            pltpu.sync_copy(m_hbm.at[pl.ds(TAIL_BASE, ACC_TAIL)],
                            acc.at[pl.ds(TAIL_BASE, ACC_TAIL)])

    @pl.when(c != 0)
    def _():
        pltpu.sync_copy(z_hbm.at[pl.ds(base, SUB_ROWS)],
                        acc.at[pl.ds(base, SUB_ROWS)])

        @pl.when(s == NS - 1)
        def _():
            pltpu.sync_copy(z_hbm.at[pl.ds(TAIL_BASE, ACC_TAIL)],
                            acc.at[pl.ds(TAIL_BASE, ACC_TAIL)])

    plsc.subcore_barrier()

    @pl.loop(0, NBLK)
    def _(blk):
        pltpu.sync_copy(src_hbm.at[w].at[blk], src_v)
        pltpu.sync_copy(dst_hbm.at[w].at[blk], dst_v)
        pltpu.async_copy(m_hbm.at[src_v.at[0]], rows_a, sem_a)

        @pl.loop(0, IDXB - 2, step=2)
        def _(j):
            pltpu.async_copy(m_hbm.at[src_v.at[j + 1]], rows_b, sem_b)
            pltpu.make_async_copy(m_hbm.at[src_v.at[j]], rows_a, sem_a).wait()
            pltpu.sync_copy(rows_a, acc.at[dst_v.at[j]], add=True)
            pltpu.async_copy(m_hbm.at[src_v.at[j + 2]], rows_a, sem_a)
            pltpu.make_async_copy(m_hbm.at[src_v.at[j + 1]], rows_b, sem_b).wait()
            pltpu.sync_copy(rows_b, acc.at[dst_v.at[j + 1]], add=True)

        pltpu.async_copy(m_hbm.at[src_v.at[IDXB - 1]], rows_b, sem_b)
        pltpu.make_async_copy(m_hbm.at[src_v.at[IDXB - 2]], rows_a, sem_a).wait()
        pltpu.sync_copy(rows_a, acc.at[dst_v.at[IDXB - 2]], add=True)
        pltpu.make_async_copy(m_hbm.at[src_v.at[IDXB - 1]], rows_b, sem_b).wait()
        pltpu.sync_copy(rows_b, acc.at[dst_v.at[IDXB - 1]], add=True)

    plsc.subcore_barrier()
    pltpu.sync_copy(acc.at[pl.ds(base, SUB_ROWS)],
                    p_hbm.at[c].at[pl.ds(base, SUB_ROWS)])

    @pl.when(s == NS - 1)
    def _():
        pltpu.sync_copy(acc.at[pl.ds(TAIL_BASE, DUMP_TAIL)],
                        p_hbm.at[c].at[pl.ds(TAIL_BASE, DUMP_TAIL)])


@functools.cache
def _make_sc_aggregate():
    mesh = plsc.VectorSubcoreMesh(core_axis_name="c", subcore_axis_name="s")
    return pl.kernel(
        _sc_aggregate_body,
        out_type=jax.ShapeDtypeStruct((NC, N, D), jnp.float32),
        mesh=mesh,
        scratch_types=[
            pltpu.VMEM((IDXB, CH), jnp.int32),
            pltpu.VMEM((IDXB, CH), jnp.int32),
            pltpu.VMEM((CH, D), jnp.float32),
            pltpu.VMEM((CH, D), jnp.float32),
            pltpu.VMEM_SHARED((NP, D), jnp.float32),
            pltpu.SemaphoreType.DMA,
            pltpu.SemaphoreType.DMA,
        ],
    )


def _sc_aggregate(m, src3, dst3, zeros):
    m_pad = jnp.concatenate([m, jnp.zeros((NP - N, D), m.dtype)], axis=0)
    return _make_sc_aggregate()(m_pad, src3, dst3, zeros)


def _mlp_body(x_ref, w1_ref, b1_ref, w2_ref, b2_ref, out_ref):
    t = jnp.dot(x_ref[...], w1_ref[...], preferred_element_type=jnp.float32,
                precision=lax.Precision.HIGHEST)
    t = jnp.maximum(t + b1_ref[...], 0.0)
    out_ref[...] = (
        jnp.dot(t, w2_ref[...], preferred_element_type=jnp.float32,
                precision=lax.Precision.HIGHEST)
        + b2_ref[...]
    )


def _onehot_t(batch_ref):
    bi = batch_ref[...]
    rows = lax.broadcasted_iota(jnp.int32, (G, N), 0)
    return (bi == rows).astype(jnp.float32)


def _segsum(Bt, v):
    return jnp.dot(Bt, v, preferred_element_type=jnp.float32,
                   precision=lax.Precision.HIGHEST)


def _bcast(Bt, stats):
    return lax.dot_general(Bt, stats, (((0,), (0,)), ((), ())),
                           preferred_element_type=jnp.float32,
                           precision=lax.Precision.HIGHEST)


def _graph_norm(Bt, cnt, hi, gw, gb, gm):
    mean = _segsum(Bt, hi) / cnt
    xc = hi - gm * _bcast(Bt, mean)
    var = _segsum(Bt, xc * xc) / cnt
    r = lax.rsqrt(var + 1e-5)
    rb = _bcast(Bt, r)
    return jnp.maximum(xc * rb * gw + gb, 0.0)


def _psum_body(p_ref, out_ref):
    out_ref[...] = p_ref[0] + p_ref[1]


def _norm_body(residual, hi_ref, h_ref, batch_ref, gw_ref, gb_ref, gm_ref,
               outh_ref):
    Bt = _onehot_t(batch_ref)
    cnt = jnp.maximum(jnp.sum(Bt, axis=1), 1.0)[:, None]
    hi = hi_ref[...]
    y = _graph_norm(Bt, cnt, hi, gw_ref[...], gb_ref[...], gm_ref[...])
    if residual:
        y = y + h_ref[...]
    outh_ref[...] = y


def _final_body(hi_ref, h_ref, batch_ref, gw_ref, gb_ref, gm_ref,
                lw_ref, lb_ref, out_ref):
    Bt = _onehot_t(batch_ref)
    cnt = jnp.maximum(jnp.sum(Bt, axis=1), 1.0)[:, None]
    hi = hi_ref[...]
    y = _graph_norm(Bt, cnt, hi, gw_ref[...], gb_ref[...], gm_ref[...])
    y = y + h_ref[...]
    pooled = _segsum(Bt, y) / cnt
    out_ref[...] = (
        jnp.dot(pooled, lw_ref[...], preferred_element_type=jnp.float32,
                precision=lax.Precision.HIGHEST)
        + lb_ref[...]
    )


_f32 = jnp.float32


def _mlp(x, w1, b1, w2, b2):
    return pl.pallas_call(
        _mlp_body,
        out_shape=jax.ShapeDtypeStruct((N, D), _f32),
    )(x, w1, b1, w2, b2)


def _psum(p):
    return pl.pallas_call(
        _psum_body,
        out_shape=jax.ShapeDtypeStruct((N, D), _f32),
    )(p)


def _norm(residual, hi, h, batch2, gw, gb, gm):
    return pl.pallas_call(
        functools.partial(_norm_body, residual),
        out_shape=jax.ShapeDtypeStruct((N, D), _f32),
    )(hi, h, batch2, gw, gb, gm)


def _final(hi, h, batch2, gw, gb, gm, lw, lb):
    return pl.pallas_call(
        _final_body,
        out_shape=jax.ShapeDtypeStruct((G, 1), _f32),
    )(hi, h, batch2, gw, gb, gm, lw, lb)


def kernel(x, edge_index, batch,
           cW1_1, cb1_1, cW2_1, cb2_1, gw_1, gb_1, gm_1,
           cW1_2, cb1_2, cW2_2, cb2_2, gw_2, gb_2, gm_2,
           cW1_3, cb1_3, cW2_3, cb2_3, gw_3, gb_3, gm_3,
           cW1_4, cb1_4, cW2_4, cb2_4, gw_4, gb_4, gm_4,
           cW1_5, cb1_5, cW2_5, cb2_5, gw_5, gb_5, gm_5,
           lin_W, lin_b):
    npad = EPAD - E
    pad_idx = (N + (jnp.arange(npad, dtype=jnp.int32) % (NP - N)))
    src3 = jnp.concatenate([edge_index[0], pad_idx]).reshape(NW, NBLK, IDXB, CH)
    dst3 = jnp.concatenate([edge_index[1], pad_idx]).reshape(NW, NBLK, IDXB, CH)
    zeros = jnp.zeros((NP, D), _f32)
    batch2 = batch.reshape(1, N)
    r2 = lambda v: v.reshape(1, -1)

    layers = [
        (cW1_1, r2(cb1_1), cW2_1, r2(cb2_1), r2(gw_1), r2(gb_1), r2(gm_1)),
        (cW1_2, r2(cb1_2), cW2_2, r2(cb2_2), r2(gw_2), r2(gb_2), r2(gm_2)),
        (cW1_3, r2(cb1_3), cW2_3, r2(cb2_3), r2(gw_3), r2(gb_3), r2(gm_3)),
        (cW1_4, r2(cb1_4), cW2_4, r2(cb2_4), r2(gw_4), r2(gb_4), r2(gm_4)),
        (cW1_5, r2(cb1_5), cW2_5, r2(cb2_5), r2(gw_5), r2(gb_5), r2(gm_5)),
    ]

    m = _mlp(x, layers[0][0], layers[0][1], layers[0][2], layers[0][3])
    h = x
    for i in range(5):
        p = _sc_aggregate(m, src3, dst3, zeros)
        hi = _psum(p)
        gw, gb, gm = layers[i][4], layers[i][5], layers[i][6]
        if i < 4:
            h = _norm(i > 0, hi, h, batch2, gw, gb, gm)
            m = _mlp(h, layers[i + 1][0], layers[i + 1][1],
                     layers[i + 1][2], layers[i + 1][3])
        else:
            out = _final(hi, h, batch2, gw, gb, gm, lin_W, r2(lin_b))
    return out


# speedup vs baseline: 1.0006x; 1.0006x over previous
import functools

import jax
import jax.numpy as jnp
from jax import lax
from jax.experimental import pallas as pl
from jax.experimental.pallas import tpu as pltpu
from jax.experimental.pallas import tpu_sc as plsc

N = 10000
E = 320000
D = 128
H = 64
G = 64

NC = 2
NS = 16
NW = NC * NS
CH = 128
NCHUNK = 80
EPW = NCHUNK * CH
EPAD = NW * EPW
NP = N + 8
IDXB = 10
NBLK = NCHUNK // IDXB
SUB_ROWS = 624
TAIL_BASE = NS * SUB_ROWS
ACC_TAIL = NP - TAIL_BASE
DUMP_TAIL = N - TAIL_BASE

def _sc_aggregate_body(m_hbm, src_hbm, dst_hbm, z_hbm, p_hbm,
                       src_v, dst_v, rows_a, rows_b, acc, sem_a, sem_b):
    c = lax.axis_index("c")
    s = lax.axis_index("s")
    w = s * NC + c
    base = s * SUB_ROWS

    @pl.when(c == 0)
    def _():
        pltpu.sync_copy(m_hbm.at[pl.ds(base, SUB_ROWS)],
                        acc.at[pl.ds(base, SUB_ROWS)])

        @pl.when(s == NS - 1)
        def _():
            pltpu.sync_copy(m_hbm.at[pl.ds(TAIL_BASE, ACC_TAIL)],
                            acc.at[pl.ds(TAIL_BASE, ACC_TAIL)])

    @pl.when(c != 0)
    def _():
        pltpu.sync_copy(z_hbm.at[pl.ds(base, SUB_ROWS)],
                        acc.at[pl.ds(base, SUB_ROWS)])

        @pl.when(s == NS - 1)
        def _():
            pltpu.sync_copy(z_hbm.at[pl.ds(TAIL_BASE, ACC_TAIL)],
                            acc.at[pl.ds(TAIL_BASE, ACC_TAIL)])

    plsc.subcore_barrier()

    @pl.loop(0, NBLK)
    def _(blk):
        pltpu.sync_copy(src_hbm.at[w].at[blk], src_v)
        pltpu.sync_copy(dst_hbm.at[w].at[blk], dst_v)
        pltpu.async_copy(m_hbm.at[src_v.at[0]], rows_a, sem_a)

        @pl.loop(0, IDXB - 2, step=2)
        def _(j):
            pltpu.async_copy(m_hbm.at[src_v.at[j + 1]], rows_b, sem_b)
            pltpu.make_async_copy(m_hbm.at[src_v.at[j]], rows_a, sem_a).wait()
            pltpu.sync_copy(rows_a, acc.at[dst_v.at[j]], add=True)
            pltpu.async_copy(m_hbm.at[src_v.at[j + 2]], rows_a, sem_a)
            pltpu.make_async_copy(m_hbm.at[src_v.at[j + 1]], rows_b, sem_b).wait()
            pltpu.sync_copy(rows_b, acc.at[dst_v.at[j + 1]], add=True)

        pltpu.async_copy(m_hbm.at[src_v.at[IDXB - 1]], rows_b, sem_b)
        pltpu.make_async_copy(m_hbm.at[src_v.at[IDXB - 2]], rows_a, sem_a).wait()
        pltpu.sync_copy(rows_a, acc.at[dst_v.at[IDXB - 2]], add=True)
        pltpu.make_async_copy(m_hbm.at[src_v.at[IDXB - 1]], rows_b, sem_b).wait()
        pltpu.sync_copy(rows_b, acc.at[dst_v.at[IDXB - 1]], add=True)

    plsc.subcore_barrier()
    pltpu.sync_copy(acc.at[pl.ds(base, SUB_ROWS)],
                    p_hbm.at[c].at[pl.ds(base, SUB_ROWS)])

    @pl.when(s == NS - 1)
    def _():
        pltpu.sync_copy(acc.at[pl.ds(TAIL_BASE, DUMP_TAIL)],
                        p_hbm.at[c].at[pl.ds(TAIL_BASE, DUMP_TAIL)])


@functools.cache
def _make_sc_aggregate():
    mesh = plsc.VectorSubcoreMesh(core_axis_name="c", subcore_axis_name="s")
    return pl.kernel(
        _sc_aggregate_body,
        out_type=jax.ShapeDtypeStruct((NC, N, D), jnp.float32),
        mesh=mesh,
        scratch_types=[
            pltpu.VMEM((IDXB, CH), jnp.int32),
            pltpu.VMEM((IDXB, CH), jnp.int32),
            pltpu.VMEM((CH, D), jnp.float32),
            pltpu.VMEM((CH, D), jnp.float32),
            pltpu.VMEM_SHARED((NP, D), jnp.float32),
            pltpu.SemaphoreType.DMA,
            pltpu.SemaphoreType.DMA,
        ],
    )


def _sc_aggregate(m, src3, dst3, zeros):
    m_pad = jnp.concatenate([m, jnp.zeros((NP - N, D), m.dtype)], axis=0)
    return _make_sc_aggregate()(m_pad, src3, dst3, zeros)


def _mlp_body(x_ref, w1_ref, b1_ref, w2_ref, b2_ref, out_ref):
    t = jnp.dot(x_ref[...], w1_ref[...], preferred_element_type=jnp.float32,
                precision=lax.Precision.HIGHEST)
    t = jnp.maximum(t + b1_ref[...], 0.0)
    out_ref[...] = (
        jnp.dot(t, w2_ref[...], preferred_element_type=jnp.float32,
                precision=lax.Precision.HIGHEST)
        + b2_ref[...]
    )


def _onehot_t(batch_ref):
    bi = batch_ref[...]
    rows = lax.broadcasted_iota(jnp.int32, (G, N), 0)
    return (bi == rows).astype(jnp.float32)


def _segsum(Bt, v):
    return jnp.dot(Bt, v, preferred_element_type=jnp.float32,
                   precision=lax.Precision.HIGHEST)


def _bcast(Bt, stats):
    return lax.dot_general(Bt, stats, (((0,), (0,)), ((), ())),
                           preferred_element_type=jnp.float32,
                           precision=lax.Precision.HIGHEST)


def _graph_norm(Bt, cnt, hi, gw, gb, gm):
    mean = _segsum(Bt, hi) / cnt
    xc = hi - gm * _bcast(Bt, mean)
    var = _segsum(Bt, xc * xc) / cnt
    r = lax.rsqrt(var + 1e-5)
    rb = _bcast(Bt, r)
    return jnp.maximum(xc * rb * gw + gb, 0.0)


def _psum_body(p_ref, out_ref):
    out_ref[...] = p_ref[0] + p_ref[1]


def _norm_body(residual, hi_ref, h_ref, batch_ref, gw_ref, gb_ref, gm_ref,
               outh_ref):
    Bt = _onehot_t(batch_ref)
    cnt = jnp.maximum(jnp.sum(Bt, axis=1), 1.0)[:, None]
    hi = hi_ref[...]
    y = _graph_norm(Bt, cnt, hi, gw_ref[...], gb_ref[...], gm_ref[...])
    if residual:
        y = y + h_ref[...]
    outh_ref[...] = y


def _final_body(hi_ref, h_ref, batch_ref, gw_ref, gb_ref, gm_ref,
                lw_ref, lb_ref, out_ref):
    Bt = _onehot_t(batch_ref)
    cnt = jnp.maximum(jnp.sum(Bt, axis=1), 1.0)[:, None]
    hi = hi_ref[...]
    y = _graph_norm(Bt, cnt, hi, gw_ref[...], gb_ref[...], gm_ref[...])
    y = y + h_ref[...]
    pooled = _segsum(Bt, y) / cnt
    out_ref[...] = (
        jnp.dot(pooled, lw_ref[...], preferred_element_type=jnp.float32,
                precision=lax.Precision.HIGHEST)
        + lb_ref[...]
    )


_f32 = jnp.float32


def _mlp(x, w1, b1, w2, b2):
    return pl.pallas_call(
        _mlp_body,
        out_shape=jax.ShapeDtypeStruct((N, D), _f32),
    )(x, w1, b1, w2, b2)


def _psum(p):
    return pl.pallas_call(
        _psum_body,
        out_shape=jax.ShapeDtypeStruct((N, D), _f32),
    )(p)


def _norm(residual, hi, h, batch2, gw, gb, gm):
    return pl.pallas_call(
        functools.partial(_norm_body, residual),
        out_shape=jax.ShapeDtypeStruct((N, D), _f32),
    )(hi, h, batch2, gw, gb, gm)


def _final(hi, h, batch2, gw, gb, gm, lw, lb):
    return pl.pallas_call(
        _final_body,
        out_shape=jax.ShapeDtypeStruct((G, 1), _f32),
    )(hi, h, batch2, gw, gb, gm, lw, lb)


def kernel(x, edge_index, batch,
           cW1_1, cb1_1, cW2_1, cb2_1, gw_1, gb_1, gm_1,
           cW1_2, cb1_2, cW2_2, cb2_2, gw_2, gb_2, gm_2,
           cW1_3, cb1_3, cW2_3, cb2_3, gw_3, gb_3, gm_3,
           cW1_4, cb1_4, cW2_4, cb2_4, gw_4, gb_4, gm_4,
           cW1_5, cb1_5, cW2_5, cb2_5, gw_5, gb_5, gm_5,
           lin_W, lin_b):
    npad = EPAD - E
    pad_src = N + (jnp.arange(npad, dtype=jnp.int32) % (NP - N))
    pad_dst = (jnp.arange(npad, dtype=jnp.int32) * 13) % N
    src3 = jnp.concatenate([edge_index[0], pad_src]).reshape(NW, NBLK, IDXB, CH)
    dst3 = jnp.concatenate([edge_index[1], pad_dst]).reshape(NW, NBLK, IDXB, CH)
    zeros = jnp.zeros((NP, D), _f32)
    batch2 = batch.reshape(1, N)
    r2 = lambda v: v.reshape(1, -1)

    layers = [
        (cW1_1, r2(cb1_1), cW2_1, r2(cb2_1), r2(gw_1), r2(gb_1), r2(gm_1)),
        (cW1_2, r2(cb1_2), cW2_2, r2(cb2_2), r2(gw_2), r2(gb_2), r2(gm_2)),
        (cW1_3, r2(cb1_3), cW2_3, r2(cb2_3), r2(gw_3), r2(gb_3), r2(gm_3)),
        (cW1_4, r2(cb1_4), cW2_4, r2(cb2_4), r2(gw_4), r2(gb_4), r2(gm_4)),
        (cW1_5, r2(cb1_5), cW2_5, r2(cb2_5), r2(gw_5), r2(gb_5), r2(gm_5)),
    ]

    m = _mlp(x, layers[0][0], layers[0][1], layers[0][2], layers[0][3])
    h = x
    for i in range(5):
        p = _sc_aggregate(m, src3, dst3, zeros)
        hi = _psum(p)
        gw, gb, gm = layers[i][4], layers[i][5], layers[i][6]
        if i < 4:
            h = _norm(i > 0, hi, h, batch2, gw, gb, gm)
            m = _mlp(h, layers[i + 1][0], layers[i + 1][1],
                     layers[i + 1][2], layers[i + 1][3])
        else:
            out = _final(hi, h, batch2, gw, gb, gm, lin_W, r2(lin_b))
    return out


# speedup vs baseline: 1.3131x; 1.3122x over previous
import functools

import jax
import jax.numpy as jnp
from jax import lax
from jax.experimental import pallas as pl
from jax.experimental.pallas import tpu as pltpu
from jax.experimental.pallas import tpu_sc as plsc

N = 10000
E = 320000
D = 128
H = 64
G = 64

NC = 2
NS = 16
NW = NC * NS
EPW = E // NW
CH = 80
NCHUNK = EPW // CH
IDXB = 25
NBLK = NCHUNK // IDXB
SUB_ROWS = 624
TAIL_BASE = NS * SUB_ROWS
TAIL_ROWS = N - TAIL_BASE

_f32 = jnp.float32
_bf16 = jnp.bfloat16


def _sc_aggregate_body(m_hbm, src_hbm, dst_hbm, z_hbm, p_hbm,
                       src_v, dst_v, rows_a, rows_b, acc, sem_a, sem_b):
    c = lax.axis_index("c")
    s = lax.axis_index("s")
    w = s * NC + c
    base = s * SUB_ROWS

    @pl.when(c == 0)
    def _():
        pltpu.sync_copy(m_hbm.at[pl.ds(base, SUB_ROWS)],
                        acc.at[pl.ds(base, SUB_ROWS)])

        @pl.when(s == NS - 1)
        def _():
            pltpu.sync_copy(m_hbm.at[pl.ds(TAIL_BASE, TAIL_ROWS)],
                            acc.at[pl.ds(TAIL_BASE, TAIL_ROWS)])

    @pl.when(c != 0)
    def _():
        pltpu.sync_copy(z_hbm.at[pl.ds(base, SUB_ROWS)],
                        acc.at[pl.ds(base, SUB_ROWS)])

        @pl.when(s == NS - 1)
        def _():
            pltpu.sync_copy(z_hbm.at[pl.ds(TAIL_BASE, TAIL_ROWS)],
                            acc.at[pl.ds(TAIL_BASE, TAIL_ROWS)])

    plsc.subcore_barrier()

    @pl.loop(0, NBLK)
    def _(blk):
        pltpu.sync_copy(src_hbm.at[w].at[blk], src_v)
        pltpu.sync_copy(dst_hbm.at[w].at[blk], dst_v)
        pltpu.async_copy(m_hbm.at[src_v.at[0]], rows_a, sem_a)

        @pl.loop(0, IDXB - 1, step=2)
        def _(j):
            pltpu.async_copy(m_hbm.at[src_v.at[j + 1]], rows_b, sem_b)
            pltpu.make_async_copy(m_hbm.at[src_v.at[j]], rows_a, sem_a).wait()
            pltpu.sync_copy(rows_a, acc.at[dst_v.at[j]], add=True)
            pltpu.async_copy(m_hbm.at[src_v.at[j + 2]], rows_a, sem_a)
            pltpu.make_async_copy(m_hbm.at[src_v.at[j + 1]], rows_b, sem_b).wait()
            pltpu.sync_copy(rows_b, acc.at[dst_v.at[j + 1]], add=True)

        pltpu.make_async_copy(m_hbm.at[src_v.at[IDXB - 1]], rows_a, sem_a).wait()
        pltpu.sync_copy(rows_a, acc.at[dst_v.at[IDXB - 1]], add=True)

    plsc.subcore_barrier()
    pltpu.sync_copy(acc.at[pl.ds(base, SUB_ROWS)],
                    p_hbm.at[c].at[pl.ds(base, SUB_ROWS)])

    @pl.when(s == NS - 1)
    def _():
        pltpu.sync_copy(acc.at[pl.ds(TAIL_BASE, TAIL_ROWS)],
                        p_hbm.at[c].at[pl.ds(TAIL_BASE, TAIL_ROWS)])


@functools.cache
def _make_sc_aggregate():
    mesh = plsc.VectorSubcoreMesh(core_axis_name="c", subcore_axis_name="s")
    return pl.kernel(
        _sc_aggregate_body,
        out_type=jax.ShapeDtypeStruct((NC, N, D), _f32),
        mesh=mesh,
        scratch_types=[
            pltpu.VMEM((IDXB, CH), jnp.int32),
            pltpu.VMEM((IDXB, CH), jnp.int32),
            pltpu.VMEM((CH, D), _f32),
            pltpu.VMEM((CH, D), _f32),
            pltpu.VMEM_SHARED((N, D), _f32),
            pltpu.SemaphoreType.DMA,
            pltpu.SemaphoreType.DMA,
        ],
    )


def _sc_aggregate(m, src4, dst4, zeros):
    return _make_sc_aggregate()(m, src4, dst4, zeros)


def _split(v):
    hi = v.astype(_bf16)
    lo = (v - hi.astype(_f32)).astype(_bf16)
    return hi, lo


def _segsum(Bt_b, v):
    hi, lo = _split(v)
    return (jnp.dot(Bt_b, hi, preferred_element_type=_f32)
            + jnp.dot(Bt_b, lo, preferred_element_type=_f32))


_DN_BCAST = (((0,), (0,)), ((), ()))


def _bcast(Bt_b, stats):
    hi, lo = _split(stats)
    return (lax.dot_general(Bt_b, hi, _DN_BCAST, preferred_element_type=_f32)
            + lax.dot_general(Bt_b, lo, _DN_BCAST,
                              preferred_element_type=_f32))


def _dot3(x, w):
    xh, xl = _split(x)
    wh, wl = _split(w)
    return (jnp.dot(xh, wh, preferred_element_type=_f32)
            + jnp.dot(xh, wl, preferred_element_type=_f32)
            + jnp.dot(xl, wh, preferred_element_type=_f32))


def _onehot_t(batch_ref):
    bi = batch_ref[...]
    rows = lax.broadcasted_iota(jnp.int32, (G, N), 0)
    return (bi == rows).astype(_f32)


def _graph_norm(Bt_b, cnt, hi, gw, gb, gm):
    mean = _segsum(Bt_b, hi) / cnt
    xc = hi - gm * _bcast(Bt_b, mean)
    var = _segsum(Bt_b, xc * xc) / cnt
    r = lax.rsqrt(var + 1e-5)
    rb = _bcast(Bt_b, r)
    return jnp.maximum(xc * rb * gw + gb, 0.0)


def _mlp(y, w1, b1, w2, b2):
    t = jnp.maximum(_dot3(y, w1) + b1, 0.0)
    return _dot3(t, w2) + b2


def _mlp_body(x_ref, w1_ref, b1_ref, w2_ref, b2_ref, out_ref):
    out_ref[...] = _mlp(x_ref[...], w1_ref[...], b1_ref[...],
                        w2_ref[...], b2_ref[...])


def _layer_body(residual, p_ref, h_ref, batch_ref, gw_ref, gb_ref, gm_ref,
                w1_ref, b1_ref, w2_ref, b2_ref, outh_ref, outm_ref):
    Bt = _onehot_t(batch_ref)
    cnt = jnp.maximum(jnp.sum(Bt, axis=1), 1.0)[:, None]
    Bt_b = Bt.astype(_bf16)
    hi = p_ref[0] + p_ref[1]
    y = _graph_norm(Bt_b, cnt, hi, gw_ref[...], gb_ref[...], gm_ref[...])
    if residual:
        y = y + h_ref[...]
    outh_ref[...] = y
    outm_ref[...] = _mlp(y, w1_ref[...], b1_ref[...], w2_ref[...], b2_ref[...])


def _final_body(p_ref, h_ref, batch_ref, gw_ref, gb_ref, gm_ref,
                lw_ref, lb_ref, out_ref):
    Bt = _onehot_t(batch_ref)
    cnt = jnp.maximum(jnp.sum(Bt, axis=1), 1.0)[:, None]
    Bt_b = Bt.astype(_bf16)
    hi = p_ref[0] + p_ref[1]
    y = _graph_norm(Bt_b, cnt, hi, gw_ref[...], gb_ref[...], gm_ref[...])
    y = y + h_ref[...]
    pooled = _segsum(Bt_b, y) / cnt
    out_ref[...] = _dot3(pooled, lw_ref[...]) + lb_ref[...]


def _mlp_call(x, w1, b1, w2, b2):
    return pl.pallas_call(
        _mlp_body,
        out_shape=jax.ShapeDtypeStruct((N, D), _f32),
    )(x, w1, b1, w2, b2)


def _layer_call(residual, p, h, batch2, gw, gb, gm, w1, b1, w2, b2):
    return pl.pallas_call(
        functools.partial(_layer_body, residual),
        out_shape=(jax.ShapeDtypeStruct((N, D), _f32),
                   jax.ShapeDtypeStruct((N, D), _f32)),
    )(p, h, batch2, gw, gb, gm, w1, b1, w2, b2)


def _final_call(p, h, batch2, gw, gb, gm, lw, lb):
    return pl.pallas_call(
        _final_body,
        out_shape=jax.ShapeDtypeStruct((G, 1), _f32),
    )(p, h, batch2, gw, gb, gm, lw, lb)


def kernel(x, edge_index, batch,
           cW1_1, cb1_1, cW2_1, cb2_1, gw_1, gb_1, gm_1,
           cW1_2, cb1_2, cW2_2, cb2_2, gw_2, gb_2, gm_2,
           cW1_3, cb1_3, cW2_3, cb2_3, gw_3, gb_3, gm_3,
           cW1_4, cb1_4, cW2_4, cb2_4, gw_4, gb_4, gm_4,
           cW1_5, cb1_5, cW2_5, cb2_5, gw_5, gb_5, gm_5,
           lin_W, lin_b):
    src4 = edge_index[0].reshape(NW, NBLK, IDXB, CH)
    dst4 = edge_index[1].reshape(NW, NBLK, IDXB, CH)
    zeros = jnp.zeros((N, D), _f32)
    batch2 = batch.reshape(1, N)
    r2 = lambda v: v.reshape(1, -1)

    layers = [
        (cW1_1, r2(cb1_1), cW2_1, r2(cb2_1), r2(gw_1), r2(gb_1), r2(gm_1)),
        (cW1_2, r2(cb1_2), cW2_2, r2(cb2_2), r2(gw_2), r2(gb_2), r2(gm_2)),
        (cW1_3, r2(cb1_3), cW2_3, r2(cb2_3), r2(gw_3), r2(gb_3), r2(gm_3)),
        (cW1_4, r2(cb1_4), cW2_4, r2(cb2_4), r2(gw_4), r2(gb_4), r2(gm_4)),
        (cW1_5, r2(cb1_5), cW2_5, r2(cb2_5), r2(gw_5), r2(gb_5), r2(gm_5)),
    ]

    m = _mlp_call(x, layers[0][0], layers[0][1], layers[0][2], layers[0][3])
    h = x
    for i in range(5):
        p = _sc_aggregate(m, src4, dst4, zeros)
        gw, gb, gm = layers[i][4], layers[i][5], layers[i][6]
        if i < 4:
            h, m = _layer_call(i > 0, p, h, batch2, gw, gb, gm,
                               layers[i + 1][0], layers[i + 1][1],
                               layers[i + 1][2], layers[i + 1][3])
        else:
            out = _final_call(p, h, batch2, gw, gb, gm, lin_W, r2(lin_b))
    return out


# speedup vs baseline: 1.4624x; 1.1137x over previous
import functools

import jax
import jax.numpy as jnp
from jax import lax
from jax.experimental import pallas as pl
from jax.experimental.pallas import tpu as pltpu
from jax.experimental.pallas import tpu_sc as plsc

N = 10000
E = 320000
D = 128
H = 64
G = 64

NC = 2
NS = 16
NW = NC * NS
EPW = E // NW
CH = 80
NCHUNK = EPW // CH
IDXB = 25
NBLK = NCHUNK // IDXB
SUB_ROWS = 624
TAIL_BASE = NS * SUB_ROWS
TAIL_ROWS = N - TAIL_BASE

_f32 = jnp.float32
_bf16 = jnp.bfloat16


def _sc_aggregate_body(m_hbm, src_hbm, dst_hbm, z_hbm, p_hbm,
                       src_v, dst_v, r0, r1, r2, acc,
                       sg0, sg1, sg2, ss0, ss1, ss2):
    c = lax.axis_index("c")
    s = lax.axis_index("s")
    w = s * NC + c
    base = s * SUB_ROWS

    @pl.when(c == 0)
    def _():
        pltpu.sync_copy(m_hbm.at[pl.ds(base, SUB_ROWS)],
                        acc.at[pl.ds(base, SUB_ROWS)])

        @pl.when(s == NS - 1)
        def _():
            pltpu.sync_copy(m_hbm.at[pl.ds(TAIL_BASE, TAIL_ROWS)],
                            acc.at[pl.ds(TAIL_BASE, TAIL_ROWS)])

    @pl.when(c != 0)
    def _():
        pltpu.sync_copy(z_hbm.at[pl.ds(base, SUB_ROWS)],
                        acc.at[pl.ds(base, SUB_ROWS)])

        @pl.when(s == NS - 1)
        def _():
            pltpu.sync_copy(z_hbm.at[pl.ds(TAIL_BASE, TAIL_ROWS)],
                            acc.at[pl.ds(TAIL_BASE, TAIL_ROWS)])

    plsc.subcore_barrier()

    bufs = ((r0, sg0, ss0), (r1, sg1, ss1), (r2, sg2, ss2))

    def _wait_gather(k, rb, sgb):
        pltpu.make_async_copy(m_hbm.at[src_v.at[k]], rb, sgb).wait()

    def _wait_scatter(rb, ssb):
        pltpu.make_async_copy(rb, acc.at[pl.ds(0, CH)], ssb).wait()

    @pl.loop(0, NBLK)
    def _(blk):
        pltpu.sync_copy(src_hbm.at[w].at[blk], src_v)
        pltpu.sync_copy(dst_hbm.at[w].at[blk], dst_v)
        pltpu.async_copy(m_hbm.at[src_v.at[0]], r0, sg0)
        pltpu.async_copy(m_hbm.at[src_v.at[1]], r1, sg1)

        @pl.loop(0, IDXB - 1, step=3)
        def _(j):
            for b, (rb, sgb, ssb) in enumerate(bufs):
                k = j + b
                nb, (rn, sgn, ssn) = (b + 2) % 3, bufs[(b + 2) % 3]
                _wait_gather(k, rb, sgb)
                pltpu.async_copy(rb, acc.at[dst_v.at[k]], ssb, add=True)

                @pl.when(k + 2 <= IDXB - 1)
                def _():
                    @pl.when(k > 0)
                    def _():
                        _wait_scatter(rn, ssn)

                    pltpu.async_copy(m_hbm.at[src_v.at[k + 2]], rn, sgn)

        _wait_gather(IDXB - 1, r0, sg0)
        pltpu.async_copy(r0, acc.at[dst_v.at[IDXB - 1]], ss0, add=True)
        _wait_scatter(r1, ss1)
        _wait_scatter(r2, ss2)
        _wait_scatter(r0, ss0)

    plsc.subcore_barrier()
    pltpu.sync_copy(acc.at[pl.ds(base, SUB_ROWS)],
                    p_hbm.at[c].at[pl.ds(base, SUB_ROWS)])

    @pl.when(s == NS - 1)
    def _():
        pltpu.sync_copy(acc.at[pl.ds(TAIL_BASE, TAIL_ROWS)],
                        p_hbm.at[c].at[pl.ds(TAIL_BASE, TAIL_ROWS)])


@functools.cache
def _make_sc_aggregate():
    mesh = plsc.VectorSubcoreMesh(core_axis_name="c", subcore_axis_name="s")
    return pl.kernel(
        _sc_aggregate_body,
        out_type=jax.ShapeDtypeStruct((NC, N, D), _f32),
        mesh=mesh,
        scratch_types=[
            pltpu.VMEM((IDXB, CH), jnp.int32),
            pltpu.VMEM((IDXB, CH), jnp.int32),
            pltpu.VMEM((CH, D), _f32),
            pltpu.VMEM((CH, D), _f32),
            pltpu.VMEM((CH, D), _f32),
            pltpu.VMEM_SHARED((N, D), _f32),
            pltpu.SemaphoreType.DMA,
            pltpu.SemaphoreType.DMA,
            pltpu.SemaphoreType.DMA,
            pltpu.SemaphoreType.DMA,
            pltpu.SemaphoreType.DMA,
            pltpu.SemaphoreType.DMA,
        ],
    )


def _sc_aggregate(m, src4, dst4, zeros):
    return _make_sc_aggregate()(m, src4, dst4, zeros)


def _split(v):
    hi = v.astype(_bf16)
    lo = (v - hi.astype(_f32)).astype(_bf16)
    return hi, lo


def _segsum(Bt_b, v):
    hi, lo = _split(v)
    return (jnp.dot(Bt_b, hi, preferred_element_type=_f32)
            + jnp.dot(Bt_b, lo, preferred_element_type=_f32))


_DN_BCAST = (((0,), (0,)), ((), ()))


def _bcast(Bt_b, stats):
    hi, lo = _split(stats)
    return (lax.dot_general(Bt_b, hi, _DN_BCAST, preferred_element_type=_f32)
            + lax.dot_general(Bt_b, lo, _DN_BCAST,
                              preferred_element_type=_f32))


def _dot3(x, w):
    xh, xl = _split(x)
    wh, wl = _split(w)
    return (jnp.dot(xh, wh, preferred_element_type=_f32)
            + jnp.dot(xh, wl, preferred_element_type=_f32)
            + jnp.dot(xl, wh, preferred_element_type=_f32))


def _onehot_t(batch_ref):
    bi = batch_ref[...]
    rows = lax.broadcasted_iota(jnp.int32, (G, N), 0)
    return (bi == rows).astype(_f32)


def _graph_norm(Bt_b, cnt, hi, gw, gb, gm):
    mean = _segsum(Bt_b, hi) / cnt
    xc = hi - gm * _bcast(Bt_b, mean)
    var = _segsum(Bt_b, xc * xc) / cnt
    r = lax.rsqrt(var + 1e-5)
    rb = _bcast(Bt_b, r)
    return jnp.maximum(xc * rb * gw + gb, 0.0)


def _mlp(y, w1, b1, w2, b2):
    t = jnp.maximum(_dot3(y, w1) + b1, 0.0)
    return _dot3(t, w2) + b2


def _mlp_body(x_ref, w1_ref, b1_ref, w2_ref, b2_ref, out_ref):
    out_ref[...] = _mlp(x_ref[...], w1_ref[...], b1_ref[...],
                        w2_ref[...], b2_ref[...])


def _layer_body(residual, p_ref, h_ref, batch_ref, gw_ref, gb_ref, gm_ref,
                w1_ref, b1_ref, w2_ref, b2_ref, outh_ref, outm_ref):
    Bt = _onehot_t(batch_ref)
    cnt = jnp.maximum(jnp.sum(Bt, axis=1), 1.0)[:, None]
    Bt_b = Bt.astype(_bf16)
    hi = p_ref[0] + p_ref[1]
    y = _graph_norm(Bt_b, cnt, hi, gw_ref[...], gb_ref[...], gm_ref[...])
    if residual:
        y = y + h_ref[...]
    outh_ref[...] = y
    outm_ref[...] = _mlp(y, w1_ref[...], b1_ref[...], w2_ref[...], b2_ref[...])


def _final_body(p_ref, h_ref, batch_ref, gw_ref, gb_ref, gm_ref,
                lw_ref, lb_ref, out_ref):
    Bt = _onehot_t(batch_ref)
    cnt = jnp.maximum(jnp.sum(Bt, axis=1), 1.0)[:, None]
    Bt_b = Bt.astype(_bf16)
    hi = p_ref[0] + p_ref[1]
    y = _graph_norm(Bt_b, cnt, hi, gw_ref[...], gb_ref[...], gm_ref[...])
    y = y + h_ref[...]
    pooled = _segsum(Bt_b, y) / cnt
    out_ref[...] = _dot3(pooled, lw_ref[...]) + lb_ref[...]


def _mlp_call(x, w1, b1, w2, b2):
    return pl.pallas_call(
        _mlp_body,
        out_shape=jax.ShapeDtypeStruct((N, D), _f32),
    )(x, w1, b1, w2, b2)


def _layer_call(residual, p, h, batch2, gw, gb, gm, w1, b1, w2, b2):
    return pl.pallas_call(
        functools.partial(_layer_body, residual),
        out_shape=(jax.ShapeDtypeStruct((N, D), _f32),
                   jax.ShapeDtypeStruct((N, D), _f32)),
    )(p, h, batch2, gw, gb, gm, w1, b1, w2, b2)


def _final_call(p, h, batch2, gw, gb, gm, lw, lb):
    return pl.pallas_call(
        _final_body,
        out_shape=jax.ShapeDtypeStruct((G, 1), _f32),
    )(p, h, batch2, gw, gb, gm, lw, lb)


def kernel(x, edge_index, batch,
           cW1_1, cb1_1, cW2_1, cb2_1, gw_1, gb_1, gm_1,
           cW1_2, cb1_2, cW2_2, cb2_2, gw_2, gb_2, gm_2,
           cW1_3, cb1_3, cW2_3, cb2_3, gw_3, gb_3, gm_3,
           cW1_4, cb1_4, cW2_4, cb2_4, gw_4, gb_4, gm_4,
           cW1_5, cb1_5, cW2_5, cb2_5, gw_5, gb_5, gm_5,
           lin_W, lin_b):
    src4 = edge_index[0].reshape(NW, NBLK, IDXB, CH)
    dst4 = edge_index[1].reshape(NW, NBLK, IDXB, CH)
    zeros = jnp.zeros((N, D), _f32)
    batch2 = batch.reshape(1, N)
    r2 = lambda v: v.reshape(1, -1)

    layers = [
        (cW1_1, r2(cb1_1), cW2_1, r2(cb2_1), r2(gw_1), r2(gb_1), r2(gm_1)),
        (cW1_2, r2(cb1_2), cW2_2, r2(cb2_2), r2(gw_2), r2(gb_2), r2(gm_2)),
        (cW1_3, r2(cb1_3), cW2_3, r2(cb2_3), r2(gw_3), r2(gb_3), r2(gm_3)),
        (cW1_4, r2(cb1_4), cW2_4, r2(cb2_4), r2(gw_4), r2(gb_4), r2(gm_4)),
        (cW1_5, r2(cb1_5), cW2_5, r2(cb2_5), r2(gw_5), r2(gb_5), r2(gm_5)),
    ]

    m = _mlp_call(x, layers[0][0], layers[0][1], layers[0][2], layers[0][3])
    h = x
    for i in range(5):
        p = _sc_aggregate(m, src4, dst4, zeros)
        gw, gb, gm = layers[i][4], layers[i][5], layers[i][6]
        if i < 4:
            h, m = _layer_call(i > 0, p, h, batch2, gw, gb, gm,
                               layers[i + 1][0], layers[i + 1][1],
                               layers[i + 1][2], layers[i + 1][3])
        else:
            out = _final_call(p, h, batch2, gw, gb, gm, lin_W, r2(lin_b))
    return out


# speedup vs baseline: 1.4895x; 1.0185x over previous
import functools

import jax
import jax.numpy as jnp
from jax import lax
from jax.experimental import pallas as pl
from jax.experimental.pallas import tpu as pltpu
from jax.experimental.pallas import tpu_sc as plsc

N = 10000
E = 320000
D = 128
H = 64
G = 64

NC = 2
NS = 16
NW = NC * NS
EPW = E // NW
CH = 80
NCHUNK = EPW // CH
IDXB = 25
NBLK = NCHUNK // IDXB
SUB_ROWS = 624
TAIL_BASE = NS * SUB_ROWS
TAIL_ROWS = N - TAIL_BASE

_f32 = jnp.float32
_bf16 = jnp.bfloat16


def _sc_aggregate_body(m_hbm, src_hbm, dst_hbm, z_hbm, p_hbm,
                       src_v, dst_v, r0, r1, r2, acc,
                       sg0, sg1, sg2, ss0, ss1, ss2, si):
    c = lax.axis_index("c")
    s = lax.axis_index("s")
    w = s * NC + c
    base = s * SUB_ROWS

    @pl.when(c == 0)
    def _():
        pltpu.async_copy(m_hbm.at[pl.ds(base, SUB_ROWS)],
                         acc.at[pl.ds(base, SUB_ROWS)], si)

        @pl.when(s == NS - 1)
        def _():
            pltpu.async_copy(m_hbm.at[pl.ds(TAIL_BASE, TAIL_ROWS)],
                             acc.at[pl.ds(TAIL_BASE, TAIL_ROWS)], si)

    @pl.when(c != 0)
    def _():
        pltpu.async_copy(z_hbm.at[pl.ds(base, SUB_ROWS)],
                         acc.at[pl.ds(base, SUB_ROWS)], si)

        @pl.when(s == NS - 1)
        def _():
            pltpu.async_copy(z_hbm.at[pl.ds(TAIL_BASE, TAIL_ROWS)],
                             acc.at[pl.ds(TAIL_BASE, TAIL_ROWS)], si)

    bufs = ((r0, sg0, ss0), (r1, sg1, ss1), (r2, sg2, ss2))

    def _wait_gather(k, rb, sgb):
        pltpu.make_async_copy(m_hbm.at[src_v.at[k]], rb, sgb).wait()

    def _wait_scatter(rb, ssb):
        pltpu.make_async_copy(rb, acc.at[pl.ds(0, CH)], ssb).wait()

    def _stage_block(blk):
        pltpu.sync_copy(src_hbm.at[w].at[blk], src_v)
        pltpu.sync_copy(dst_hbm.at[w].at[blk], dst_v)
        pltpu.async_copy(m_hbm.at[src_v.at[0]], r0, sg0)
        pltpu.async_copy(m_hbm.at[src_v.at[1]], r1, sg1)

    def _run_block():
        @pl.loop(0, IDXB - 1, step=3)
        def _(j):
            for b, (rb, sgb, ssb) in enumerate(bufs):
                k = j + b
                nb, (rn, sgn, ssn) = (b + 2) % 3, bufs[(b + 2) % 3]
                _wait_gather(k, rb, sgb)
                pltpu.async_copy(rb, acc.at[dst_v.at[k]], ssb, add=True)

                @pl.when(k + 2 <= IDXB - 1)
                def _():
                    @pl.when(k > 0)
                    def _():
                        _wait_scatter(rn, ssn)

                    pltpu.async_copy(m_hbm.at[src_v.at[k + 2]], rn, sgn)

        _wait_gather(IDXB - 1, r0, sg0)
        pltpu.async_copy(r0, acc.at[dst_v.at[IDXB - 1]], ss0, add=True)
        _wait_scatter(r1, ss1)
        _wait_scatter(r2, ss2)
        _wait_scatter(r0, ss0)

    _stage_block(0)
    pltpu.make_async_copy(m_hbm.at[pl.ds(base, SUB_ROWS)],
                          acc.at[pl.ds(base, SUB_ROWS)], si).wait()

    @pl.when(s == NS - 1)
    def _():
        pltpu.make_async_copy(m_hbm.at[pl.ds(TAIL_BASE, TAIL_ROWS)],
                              acc.at[pl.ds(TAIL_BASE, TAIL_ROWS)], si).wait()

    plsc.subcore_barrier()
    _run_block()

    @pl.loop(1, NBLK)
    def _(blk):
        _stage_block(blk)
        _run_block()

    plsc.subcore_barrier()
    pltpu.sync_copy(acc.at[pl.ds(base, SUB_ROWS)],
                    p_hbm.at[c].at[pl.ds(base, SUB_ROWS)])

    @pl.when(s == NS - 1)
    def _():
        pltpu.sync_copy(acc.at[pl.ds(TAIL_BASE, TAIL_ROWS)],
                        p_hbm.at[c].at[pl.ds(TAIL_BASE, TAIL_ROWS)])


@functools.cache
def _make_sc_aggregate():
    mesh = plsc.VectorSubcoreMesh(core_axis_name="c", subcore_axis_name="s")
    return pl.kernel(
        _sc_aggregate_body,
        out_type=jax.ShapeDtypeStruct((NC, N, D), _f32),
        mesh=mesh,
        scratch_types=[
            pltpu.VMEM((IDXB, CH), jnp.int32),
            pltpu.VMEM((IDXB, CH), jnp.int32),
            pltpu.VMEM((CH, D), _f32),
            pltpu.VMEM((CH, D), _f32),
            pltpu.VMEM((CH, D), _f32),
            pltpu.VMEM_SHARED((N, D), _f32),
            pltpu.SemaphoreType.DMA,
            pltpu.SemaphoreType.DMA,
            pltpu.SemaphoreType.DMA,
            pltpu.SemaphoreType.DMA,
            pltpu.SemaphoreType.DMA,
            pltpu.SemaphoreType.DMA,
            pltpu.SemaphoreType.DMA,
        ],
    )


def _sc_aggregate(m, src4, dst4, zeros):
    return _make_sc_aggregate()(m, src4, dst4, zeros)


def _split(v):
    hi = v.astype(_bf16)
    lo = (v - hi.astype(_f32)).astype(_bf16)
    return hi, lo


def _segsum(Bt_b, v):
    hi, lo = _split(v)
    return (jnp.dot(Bt_b, hi, preferred_element_type=_f32)
            + jnp.dot(Bt_b, lo, preferred_element_type=_f32))


_DN_BCAST = (((0,), (0,)), ((), ()))


def _bcast(Bt_b, stats):
    hi, lo = _split(stats)
    return (lax.dot_general(Bt_b, hi, _DN_BCAST, preferred_element_type=_f32)
            + lax.dot_general(Bt_b, lo, _DN_BCAST,
                              preferred_element_type=_f32))


def _dot3(x, w):
    xh, xl = _split(x)
    wh, wl = _split(w)
    return (jnp.dot(xh, wh, preferred_element_type=_f32)
            + jnp.dot(xh, wl, preferred_element_type=_f32)
            + jnp.dot(xl, wh, preferred_element_type=_f32))


def _onehot_t(batch_ref):
    bi = batch_ref[...]
    rows = lax.broadcasted_iota(jnp.int32, (G, N), 0)
    return (bi == rows).astype(_f32)


def _graph_norm(Bt_b, cnt, hi, gw, gb, gm):
    mean = _segsum(Bt_b, hi) / cnt
    xc = hi - gm * _bcast(Bt_b, mean)
    var = _segsum(Bt_b, xc * xc) / cnt
    r = lax.rsqrt(var + 1e-5)
    rb = _bcast(Bt_b, r)
    return jnp.maximum(xc * rb * gw + gb, 0.0)


def _mlp(y, w1, b1, w2, b2):
    t = jnp.maximum(_dot3(y, w1) + b1, 0.0)
    return _dot3(t, w2) + b2


def _mlp_body(x_ref, w1_ref, b1_ref, w2_ref, b2_ref, out_ref):
    out_ref[...] = _mlp(x_ref[...], w1_ref[...], b1_ref[...],
                        w2_ref[...], b2_ref[...])


def _layer_body(residual, p_ref, h_ref, batch_ref, gw_ref, gb_ref, gm_ref,
                w1_ref, b1_ref, w2_ref, b2_ref, outh_ref, outm_ref):
    Bt = _onehot_t(batch_ref)
    cnt = jnp.maximum(jnp.sum(Bt, axis=1), 1.0)[:, None]
    Bt_b = Bt.astype(_bf16)
    hi = p_ref[0] + p_ref[1]
    y = _graph_norm(Bt_b, cnt, hi, gw_ref[...], gb_ref[...], gm_ref[...])
    if residual:
        y = y + h_ref[...]
    outh_ref[...] = y
    outm_ref[...] = _mlp(y, w1_ref[...], b1_ref[...], w2_ref[...], b2_ref[...])


def _final_body(p_ref, h_ref, batch_ref, gw_ref, gb_ref, gm_ref,
                lw_ref, lb_ref, out_ref):
    Bt = _onehot_t(batch_ref)
    cnt = jnp.maximum(jnp.sum(Bt, axis=1), 1.0)[:, None]
    Bt_b = Bt.astype(_bf16)
    hi = p_ref[0] + p_ref[1]
    y = _graph_norm(Bt_b, cnt, hi, gw_ref[...], gb_ref[...], gm_ref[...])
    y = y + h_ref[...]
    pooled = _segsum(Bt_b, y) / cnt
    out_ref[...] = _dot3(pooled, lw_ref[...]) + lb_ref[...]


def _mlp_call(x, w1, b1, w2, b2):
    return pl.pallas_call(
        _mlp_body,
        out_shape=jax.ShapeDtypeStruct((N, D), _f32),
    )(x, w1, b1, w2, b2)


def _layer_call(residual, p, h, batch2, gw, gb, gm, w1, b1, w2, b2):
    return pl.pallas_call(
        functools.partial(_layer_body, residual),
        out_shape=(jax.ShapeDtypeStruct((N, D), _f32),
                   jax.ShapeDtypeStruct((N, D), _f32)),
    )(p, h, batch2, gw, gb, gm, w1, b1, w2, b2)


def _final_call(p, h, batch2, gw, gb, gm, lw, lb):
    return pl.pallas_call(
        _final_body,
        out_shape=jax.ShapeDtypeStruct((G, 1), _f32),
    )(p, h, batch2, gw, gb, gm, lw, lb)


def kernel(x, edge_index, batch,
           cW1_1, cb1_1, cW2_1, cb2_1, gw_1, gb_1, gm_1,
           cW1_2, cb1_2, cW2_2, cb2_2, gw_2, gb_2, gm_2,
           cW1_3, cb1_3, cW2_3, cb2_3, gw_3, gb_3, gm_3,
           cW1_4, cb1_4, cW2_4, cb2_4, gw_4, gb_4, gm_4,
           cW1_5, cb1_5, cW2_5, cb2_5, gw_5, gb_5, gm_5,
           lin_W, lin_b):
    src4 = edge_index[0].reshape(NW, NBLK, IDXB, CH)
    dst4 = edge_index[1].reshape(NW, NBLK, IDXB, CH)
    zeros = jnp.zeros((N, D), _f32)
    batch2 = batch.reshape(1, N)
    r2 = lambda v: v.reshape(1, -1)

    layers = [
        (cW1_1, r2(cb1_1), cW2_1, r2(cb2_1), r2(gw_1), r2(gb_1), r2(gm_1)),
        (cW1_2, r2(cb1_2), cW2_2, r2(cb2_2), r2(gw_2), r2(gb_2), r2(gm_2)),
        (cW1_3, r2(cb1_3), cW2_3, r2(cb2_3), r2(gw_3), r2(gb_3), r2(gm_3)),
        (cW1_4, r2(cb1_4), cW2_4, r2(cb2_4), r2(gw_4), r2(gb_4), r2(gm_4)),
        (cW1_5, r2(cb1_5), cW2_5, r2(cb2_5), r2(gw_5), r2(gb_5), r2(gm_5)),
    ]

    m = _mlp_call(x, layers[0][0], layers[0][1], layers[0][2], layers[0][3])
    h = x
    for i in range(5):
        p = _sc_aggregate(m, src4, dst4, zeros)
        gw, gb, gm = layers[i][4], layers[i][5], layers[i][6]
        if i < 4:
            h, m = _layer_call(i > 0, p, h, batch2, gw, gb, gm,
                               layers[i + 1][0], layers[i + 1][1],
                               layers[i + 1][2], layers[i + 1][3])
        else:
            out = _final_call(p, h, batch2, gw, gb, gm, lin_W, r2(lin_b))
    return out
